# baseline (speedup 1.0000x reference)
import jax
import jax.numpy as jnp
from jax import lax
from jax.experimental import pallas as pl
from jax.experimental.pallas import tpu as pltpu
from jax.experimental.pallas import tpu_sc as plsc

DEPTH = 1000
N = 4096
J = 26
NUM_CORES = 2
NUM_SUBCORES = 16
NW = NUM_CORES * NUM_SUBCORES
TROWS = DEPTH // 8
NSLAB = J * TROWS
BASE = NSLAB // NW
EXTRA = NSLAB - BASE * NW
NBUF = 3
LANE = 16
NITER = (BASE + 1 + NBUF - 1) // NBUF


def _onehot_sc_body(idx_hbm, onoff_hbm, tmpl_hbm, out_hbm,
                    idxrow_v, onoff_v, buf0, buf1, buf2,
                    hsem0, hsem1, hsem2, csem0, csem1, csem2):
    wid = lax.axis_index("s") * NUM_CORES + lax.axis_index("c")
    nslab = jnp.where(wid < EXTRA, BASE + 1, BASE)
    s0 = jnp.where(wid < EXTRA, wid * (BASE + 1),
                   EXTRA * (BASE + 1) + (wid - EXTRA) * BASE)
    j0 = s0 // TROWS

    pltpu.sync_copy(idx_hbm.at[pl.ds(j0 * N, 2 * N)], idxrow_v)
    pltpu.sync_copy(onoff_hbm, onoff_v)
    on_vec = onoff_v[pl.ds(0, LANE)]
    lane = lax.iota(jnp.int32, LANE)

    bufs = (buf0, buf1, buf2)
    hsems = (hsem0, hsem1, hsem2)
    csems = (csem0, csem1, csem2)

    def start_cp(b):
        pltpu.async_copy(tmpl_hbm, bufs[b], csems[b])

    def wait_cp(b):
        pltpu.make_async_copy(tmpl_hbm, bufs[b], csems[b]).wait()

    def slab_jt(s):
        j = s // TROWS
        return j, s - j * TROWS

    def scat_slab(b, s):
        j, t = slab_jt(s)
        jj = j - j0
        d0 = t * 8

        def body(k, c):
            d16 = idxrow_v[pl.ds(jj * N + k * LANE, LANE)]
            rel = d16 - d0
            m = rel.astype(jnp.uint32) < 8
            i16 = k * LANE + lane
            plsc.store_scatter(bufs[b], [rel, i16], on_vec, mask=m)
            return c
        lax.fori_loop(0, N // LANE, body, 0)

    def start_hbm(b, s):
        j, t = slab_jt(s)
        d0 = pl.multiple_of(t * 8, 8)
        pltpu.async_copy(bufs[b], out_hbm.at[j, pl.ds(d0, 8)], hsems[b])

    def wait_hbm(b):
        pltpu.make_async_copy(bufs[b], out_hbm.at[0, pl.ds(0, 8)],
                              hsems[b]).wait()

    for b in range(NBUF):
        start_cp(b)

    def step(i, c):
        for b in range(NBUF):
            g = i * NBUF + b

            @pl.when(g < nslab)
            def _():
                @pl.when(g >= 2)
                def _():
                    pb = (b + 1) % NBUF
                    wait_hbm(pb)
                    start_cp(pb)

                wait_cp(b)
                scat_slab(b, s0 + g)
                start_hbm(b, s0 + g)
        return c

    lax.fori_loop(0, NITER, step, 0)

    for b in range(NBUF):
        @pl.when((jnp.mod(nslab - 1, NBUF) == b) |
                 (jnp.mod(nslab - 2, NBUF) == b))
        def _():
            wait_hbm(b)

        @pl.when(jnp.mod(nslab - 3, NBUF) == b)
        def _():
            wait_cp(b)


def kernel(indices, on_value, off_value):
    idx_t = indices.T.astype(jnp.int32)
    idx_t = jnp.pad(idx_t, ((0, 1), (0, 0))).reshape(-1)
    onoff = jnp.concatenate([
        jnp.full((LANE,), on_value, jnp.float32),
        jnp.full((LANE,), off_value, jnp.float32),
    ])
    tmpl = jnp.full((8, N), off_value, jnp.float32)
    mesh = plsc.VectorSubcoreMesh(
        core_axis_name="c", subcore_axis_name="s",
        num_cores=NUM_CORES, num_subcores=NUM_SUBCORES)
    out = pl.kernel(
        _onehot_sc_body,
        out_type=jax.ShapeDtypeStruct((J, DEPTH, N), jnp.float32),
        mesh=mesh,
        compiler_params=pltpu.CompilerParams(
            needs_layout_passes=False, use_tc_tiling_on_sc=True),
        scratch_types=(
            [pltpu.VMEM((2 * N,), jnp.int32),
             pltpu.VMEM((2 * LANE,), jnp.float32)]
            + [pltpu.VMEM((8, N), jnp.float32)] * NBUF
            + [pltpu.SemaphoreType.DMA] * (2 * NBUF)
        ),
    )(idx_t, onoff, tmpl)
    return jnp.transpose(out, (2, 0, 1))


# speedup vs baseline: 1.7756x; 1.7756x over previous
import jax
import jax.numpy as jnp
from jax import lax
from jax.experimental import pallas as pl
from jax.experimental.pallas import tpu as pltpu
from jax.experimental.pallas import tpu_sc as plsc

DEPTH = 1000
N = 4096
J = 26
NUM_CORES = 2
NUM_SUBCORES = 16
NW = NUM_CORES * NUM_SUBCORES
TROWS = DEPTH // 8
NSLAB = J * TROWS
BASE = NSLAB // NW
EXTRA = NSLAB - BASE * NW
NBUF = 2
LANE = 16
NITER = (BASE + 1 - 2 + NBUF - 1) // NBUF


def _onehot_sc_body(idx_hbm, onoff_hbm, out_hbm,
                    idxrow_v, onoff_v, buf0, buf1, sem0, sem1):
    wid = lax.axis_index("s") * NUM_CORES + lax.axis_index("c")
    nslab = jnp.where(wid < EXTRA, BASE + 1, BASE)
    s0 = jnp.where(wid < EXTRA, wid * (BASE + 1),
                   EXTRA * (BASE + 1) + (wid - EXTRA) * BASE)
    j0 = s0 // TROWS

    pltpu.sync_copy(idx_hbm.at[pl.ds(j0 * N, 2 * N)], idxrow_v)
    pltpu.sync_copy(onoff_hbm, onoff_v)
    on_vec = onoff_v[pl.ds(0, LANE)]
    off_vec = onoff_v[pl.ds(LANE, LANE)]
    lane = lax.iota(jnp.int32, LANE)

    bufs = (buf0, buf1)
    sems = (sem0, sem1)

    def fill(buf):
        for r in range(8):
            def body(k, c):
                buf[r, pl.ds(k * LANE, LANE)] = off_vec
                return c
            lax.fori_loop(0, N // LANE, body, 0, unroll=8)

    fill(buf0)
    fill(buf1)

    def slab_jt(s):
        j = s // TROWS
        return j, s - j * TROWS

    def scan(b, s_on, s_reset=None):
        j_on, t_on = slab_jt(s_on)
        jo = (j_on - j0) * N
        do_on = t_on * 8
        if s_reset is not None:
            j_re, t_re = slab_jt(s_reset)
            jr = (j_re - j0) * N
            do_re = t_re * 8

        def body(k, c):
            i16 = k * LANE + lane
            if s_reset is not None:
                dr = idxrow_v[pl.ds(jr + k * LANE, LANE)]
                relr = dr - do_re
                mr = relr.astype(jnp.uint32) < 8
                plsc.store_scatter(bufs[b], [relr, i16], off_vec, mask=mr)
            dn = idxrow_v[pl.ds(jo + k * LANE, LANE)]
            reln = dn - do_on
            mn = reln.astype(jnp.uint32) < 8
            plsc.store_scatter(bufs[b], [reln, i16], on_vec, mask=mn)
            return c
        lax.fori_loop(0, N // LANE, body, 0, unroll=8)

    def start_hbm(b, s):
        j, t = slab_jt(s)
        d0 = pl.multiple_of(t * 8, 8)
        pltpu.async_copy(bufs[b], out_hbm.at[j, pl.ds(d0, 8)], sems[b])

    def wait_hbm(b):
        pltpu.make_async_copy(bufs[b], out_hbm.at[0, pl.ds(0, 8)],
                              sems[b]).wait()

    for b in range(NBUF):
        scan(b, s0 + b)
        start_hbm(b, s0 + b)

    def step(i, c):
        for b in range(NBUF):
            g = 2 + i * NBUF + b

            @pl.when(g < nslab)
            def _():
                wait_hbm(b)
                scan(b, s0 + g, s_reset=s0 + g - NBUF)
                start_hbm(b, s0 + g)
        return c

    lax.fori_loop(0, NITER, step, 0)

    for b in range(NBUF):
        wait_hbm(b)


def kernel(indices, on_value, off_value):
    idx_t = indices.T.astype(jnp.int32)
    idx_t = jnp.pad(idx_t, ((0, 1), (0, 0))).reshape(-1)
    onoff = jnp.concatenate([
        jnp.full((LANE,), on_value, jnp.float32),
        jnp.full((LANE,), off_value, jnp.float32),
    ])
    mesh = plsc.VectorSubcoreMesh(
        core_axis_name="c", subcore_axis_name="s",
        num_cores=NUM_CORES, num_subcores=NUM_SUBCORES)
    out = pl.kernel(
        _onehot_sc_body,
        out_type=jax.ShapeDtypeStruct((J, DEPTH, N), jnp.float32),
        mesh=mesh,
        compiler_params=pltpu.CompilerParams(
            needs_layout_passes=False, use_tc_tiling_on_sc=True),
        scratch_types=(
            [pltpu.VMEM((2 * N,), jnp.int32),
             pltpu.VMEM((2 * LANE,), jnp.float32),
             pltpu.VMEM((8, N), jnp.float32),
             pltpu.VMEM((8, N), jnp.float32)]
            + [pltpu.SemaphoreType.DMA] * NBUF
        ),
    )(idx_t, onoff)
    return jnp.transpose(out, (2, 0, 1))


# speedup vs baseline: 2.6699x; 1.5036x over previous
import jax
import jax.numpy as jnp
from jax import lax
from jax.experimental import pallas as pl
from jax.experimental.pallas import tpu as pltpu
from jax.experimental.pallas import tpu_sc as plsc

DEPTH = 1000
N = 4096
J = 26
NUM_CORES = 2
NUM_SUBCORES = 16
NW = NUM_CORES * NUM_SUBCORES
TROWS = DEPTH // 8
NSLAB = J * TROWS
BASE = NSLAB // NW
EXTRA = NSLAB - BASE * NW
NBUF = 2
LANE = 16
NITER = (BASE + 1 - 2 + NBUF - 1) // NBUF


def _onehot_sc_body(idx_hbm, onoff_hbm, out_hbm,
                    idxrow_v, onoff_v, head_v, next_v, buf0, buf1, sem0, sem1):
    wid = lax.axis_index("s") * NUM_CORES + lax.axis_index("c")
    nslab = jnp.where(wid < EXTRA, BASE + 1, BASE)
    s0 = jnp.where(wid < EXTRA, wid * (BASE + 1),
                   EXTRA * (BASE + 1) + (wid - EXTRA) * BASE)
    j0 = s0 // TROWS

    pltpu.sync_copy(idx_hbm.at[pl.ds(j0 * N, 2 * N)], idxrow_v)
    pltpu.sync_copy(onoff_hbm, onoff_v)
    on_vec = onoff_v[pl.ds(0, LANE)]
    off_vec = onoff_v[pl.ds(LANE, LANE)]
    lane = lax.iota(jnp.int32, LANE)
    zero16 = lane ^ lane
    neg1 = zero16 - 1
    lane0 = lane == 0

    def spl(x):
        return zero16 + x

    bufs = (buf0, buf1)
    sems = (sem0, sem1)

    def fill(buf):
        for r in range(8):
            def body(k, c):
                buf[r, pl.ds(k * LANE, LANE)] = off_vec
                return c
            lax.fori_loop(0, N // LANE, body, 0, unroll=8)

    fill(buf0)
    fill(buf1)

    def clear_heads(k, c):
        head_v[pl.ds(k * LANE, LANE)] = neg1
        return c
    lax.fori_loop(0, 256 // LANE, clear_heads, 0)

    def build(c):
        cN = c * N
        c128 = c * 128

        def body(e, carry):
            d = plsc.load_gather(idxrow_v, [spl(cN + e)])
            t = c128 + (d >> 3)
            h = plsc.load_gather(head_v, [t])
            plsc.store_scatter(next_v, [spl(cN + e)], h, mask=lane0)
            plsc.store_scatter(head_v, [t], spl(e), mask=lane0)
            return carry
        lax.fori_loop(0, N, body, 0)

    build(0)
    j_last = (s0 + nslab - 1) // TROWS

    @pl.when(j_last > j0)
    def _():
        build(1)

    def slab_jt(s):
        j = s // TROWS
        return j, s - j * TROWS

    def poke_slab(b, s, val):
        j, t = slab_jt(s)
        c = j - j0
        cN = c * N

        def chase(ev):
            d = plsc.load_gather(idxrow_v, [cN + ev])
            plsc.store_scatter(bufs[b], [d & 7, ev], val, mask=lane0)
            return plsc.load_gather(next_v, [cN + ev])

        e0 = plsc.load_gather(head_v, [spl(c * 128 + t)])
        lax.while_loop(lambda ev: jnp.max(ev) >= 0, chase, e0)

    def start_hbm(b, s):
        j, t = slab_jt(s)
        d0 = pl.multiple_of(t * 8, 8)
        pltpu.async_copy(bufs[b], out_hbm.at[j, pl.ds(d0, 8)], sems[b])

    def wait_hbm(b):
        pltpu.make_async_copy(bufs[b], out_hbm.at[0, pl.ds(0, 8)],
                              sems[b]).wait()

    for b in range(NBUF):
        poke_slab(b, s0 + b, on_vec)
        start_hbm(b, s0 + b)

    def step(i, c):
        for b in range(NBUF):
            g = 2 + i * NBUF + b

            @pl.when(g < nslab)
            def _():
                wait_hbm(b)
                poke_slab(b, s0 + g - NBUF, off_vec)
                poke_slab(b, s0 + g, on_vec)
                start_hbm(b, s0 + g)
        return c

    lax.fori_loop(0, NITER, step, 0)

    for b in range(NBUF):
        wait_hbm(b)


def kernel(indices, on_value, off_value):
    idx_t = indices.T.astype(jnp.int32)
    idx_t = jnp.pad(idx_t, ((0, 1), (0, 0))).reshape(-1)
    onoff = jnp.concatenate([
        jnp.full((LANE,), on_value, jnp.float32),
        jnp.full((LANE,), off_value, jnp.float32),
    ])
    mesh = plsc.VectorSubcoreMesh(
        core_axis_name="c", subcore_axis_name="s",
        num_cores=NUM_CORES, num_subcores=NUM_SUBCORES)
    out = pl.kernel(
        _onehot_sc_body,
        out_type=jax.ShapeDtypeStruct((J, DEPTH, N), jnp.float32),
        mesh=mesh,
        compiler_params=pltpu.CompilerParams(
            needs_layout_passes=False, use_tc_tiling_on_sc=True),
        scratch_types=(
            [pltpu.VMEM((2 * N,), jnp.int32),
             pltpu.VMEM((2 * LANE,), jnp.float32),
             pltpu.VMEM((256,), jnp.int32),
             pltpu.VMEM((2 * N,), jnp.int32),
             pltpu.VMEM((8, N), jnp.float32),
             pltpu.VMEM((8, N), jnp.float32)]
            + [pltpu.SemaphoreType.DMA] * NBUF
        ),
    )(idx_t, onoff)
    return jnp.transpose(out, (2, 0, 1))


# speedup vs baseline: 3.3130x; 1.2409x over previous
import jax
import jax.numpy as jnp
from jax import lax
from jax.experimental import pallas as pl
from jax.experimental.pallas import tpu as pltpu
from jax.experimental.pallas import tpu_sc as plsc

DEPTH = 1000
N = 4096
J = 26
NUM_CORES = 2
NUM_SUBCORES = 16
NW = NUM_CORES * NUM_SUBCORES
TROWS = DEPTH // 8
NSLAB = J * TROWS
BASE = NSLAB // NW
EXTRA = NSLAB - BASE * NW
NBUF = 2
LANE = 16
NITER = (BASE + 1 - 2 + NBUF - 1) // NBUF


def _onehot_sc_body(idx_hbm, onoff_hbm, out_hbm,
                    idxrow_v, onoff_v, head_v, next_v, buf0, buf1, sem0, sem1):
    wid = lax.axis_index("s") * NUM_CORES + lax.axis_index("c")
    nslab = jnp.where(wid < EXTRA, BASE + 1, BASE)
    s0 = jnp.where(wid < EXTRA, wid * (BASE + 1),
                   EXTRA * (BASE + 1) + (wid - EXTRA) * BASE)
    j0 = s0 // TROWS

    pltpu.sync_copy(idx_hbm.at[pl.ds(j0 * N, 2 * N)], idxrow_v)
    pltpu.sync_copy(onoff_hbm, onoff_v)
    on_vec = onoff_v[pl.ds(0, LANE)]
    off_vec = onoff_v[pl.ds(LANE, LANE)]
    lane = lax.iota(jnp.int32, LANE)
    zero16 = lane ^ lane
    neg1 = zero16 - 1
    lane0 = lane == 0

    def spl(x):
        return zero16 + x

    bufs = (buf0, buf1)
    sems = (sem0, sem1)

    def fill(buf):
        for r in range(8):
            def body(k, c):
                buf[r, pl.ds(k * LANE, LANE)] = off_vec
                return c
            lax.fori_loop(0, N // LANE, body, 0, unroll=8)

    fill(buf0)
    fill(buf1)

    def clear_heads(k, c):
        head_v[pl.ds(k * LANE, LANE)] = neg1
        return c
    lax.fori_loop(0, 256 // LANE, clear_heads, 0)

    def build(c):
        cN = c * N
        c128 = c * 128

        def body(e, carry):
            d = plsc.load_gather(idxrow_v, [spl(cN + e)])
            t = c128 + (d >> 3)
            h = plsc.load_gather(head_v, [t])
            plsc.store_scatter(next_v, [spl(cN + e)], h, mask=lane0)
            plsc.store_scatter(head_v, [t], spl(e), mask=lane0)
            return carry
        lax.fori_loop(0, N, body, 0)

    build(0)
    j_last = (s0 + nslab - 1) // TROWS

    @pl.when(j_last > j0)
    def _():
        build(1)

    def slab_jt(s):
        j = s // TROWS
        return j, s - j * TROWS

    def poke_slab(b, s, val):
        j, t = slab_jt(s)
        c = j - j0
        cN = c * N

        def chase(ev):
            for _ in range(4):
                live = ev >= 0
                evc = jnp.maximum(ev, 0)
                d = plsc.load_gather(idxrow_v, [cN + evc])
                plsc.store_scatter(bufs[b], [d & 7, evc], val,
                                   mask=lane0 & live)
                nxt = plsc.load_gather(next_v, [cN + evc])
                ev = jnp.where(live, nxt, ev)
            return ev

        e0 = plsc.load_gather(head_v, [spl(c * 128 + t)])
        lax.while_loop(lambda ev: jnp.max(ev) >= 0, chase, e0)

    def start_hbm(b, s):
        j, t = slab_jt(s)
        d0 = pl.multiple_of(t * 8, 8)
        pltpu.async_copy(bufs[b], out_hbm.at[j, pl.ds(d0, 8)], sems[b])

    def wait_hbm(b):
        pltpu.make_async_copy(bufs[b], out_hbm.at[0, pl.ds(0, 8)],
                              sems[b]).wait()

    for b in range(NBUF):
        poke_slab(b, s0 + b, on_vec)
        start_hbm(b, s0 + b)

    def step(i, c):
        for b in range(NBUF):
            g = 2 + i * NBUF + b

            @pl.when(g < nslab)
            def _():
                wait_hbm(b)
                poke_slab(b, s0 + g - NBUF, off_vec)
                poke_slab(b, s0 + g, on_vec)
                start_hbm(b, s0 + g)
        return c

    lax.fori_loop(0, NITER, step, 0)

    for b in range(NBUF):
        wait_hbm(b)


def kernel(indices, on_value, off_value):
    idx_t = indices.T.astype(jnp.int32)
    idx_t = jnp.pad(idx_t, ((0, 1), (0, 0))).reshape(-1)
    onoff = jnp.concatenate([
        jnp.full((LANE,), on_value, jnp.float32),
        jnp.full((LANE,), off_value, jnp.float32),
    ])
    mesh = plsc.VectorSubcoreMesh(
        core_axis_name="c", subcore_axis_name="s",
        num_cores=NUM_CORES, num_subcores=NUM_SUBCORES)
    out = pl.kernel(
        _onehot_sc_body,
        out_type=jax.ShapeDtypeStruct((J, DEPTH, N), jnp.float32),
        mesh=mesh,
        compiler_params=pltpu.CompilerParams(
            needs_layout_passes=False, use_tc_tiling_on_sc=True),
        scratch_types=(
            [pltpu.VMEM((2 * N,), jnp.int32),
             pltpu.VMEM((2 * LANE,), jnp.float32),
             pltpu.VMEM((256,), jnp.int32),
             pltpu.VMEM((2 * N,), jnp.int32),
             pltpu.VMEM((8, N), jnp.float32),
             pltpu.VMEM((8, N), jnp.float32)]
            + [pltpu.SemaphoreType.DMA] * NBUF
        ),
    )(idx_t, onoff)
    return jnp.transpose(out, (2, 0, 1))


# speedup vs baseline: 5.6038x; 1.6915x over previous
import jax
import jax.numpy as jnp
from jax import lax
from jax.experimental import pallas as pl
from jax.experimental.pallas import tpu as pltpu
from jax.experimental.pallas import tpu_sc as plsc

DEPTH = 1000
N = 4096
J = 26
NUM_CORES = 2
NUM_SUBCORES = 16
NW = NUM_CORES * NUM_SUBCORES
TROWS = DEPTH // 8
NSLAB = J * TROWS
BASE = NSLAB // NW
EXTRA = NSLAB - BASE * NW
NBUF = 2
LANE = 16
NITER = (BASE + 1 - 2 + NBUF - 1) // NBUF


def _onehot_sc_body(idx_hbm, onoff_hbm, out_hbm,
                    idxrow_v, onoff_v, head_v, next_v, buf0, buf1, sem0, sem1):
    wid = lax.axis_index("s") * NUM_CORES + lax.axis_index("c")
    nslab = jnp.where(wid < EXTRA, BASE + 1, BASE)
    s0 = jnp.where(wid < EXTRA, wid * (BASE + 1),
                   EXTRA * (BASE + 1) + (wid - EXTRA) * BASE)
    j0 = s0 // TROWS

    pltpu.sync_copy(idx_hbm.at[pl.ds(j0 * N, 2 * N)], idxrow_v)
    pltpu.sync_copy(onoff_hbm, onoff_v)
    on_vec = onoff_v[pl.ds(0, LANE)]
    off_vec = onoff_v[pl.ds(LANE, LANE)]
    lane = lax.iota(jnp.int32, LANE)
    zero16 = lane ^ lane
    neg1 = zero16 - 1
    lane0 = lane == 0

    def spl(x):
        return zero16 + x

    bufs = (buf0, buf1)
    sems = (sem0, sem1)

    def fill(buf):
        for r in range(8):
            def body(k, c):
                buf[r, pl.ds(k * LANE, LANE)] = off_vec
                return c
            lax.fori_loop(0, N // LANE, body, 0, unroll=8)

    fill(buf0)
    fill(buf1)

    def clear_heads(k, c):
        head_v[pl.ds(k * LANE, LANE)] = neg1
        return c
    lax.fori_loop(0, 2 * 128, clear_heads, 0, unroll=8)

    def build(c):
        cN = c * N
        c128 = c * 128

        def body(k, carry):
            e16 = k * LANE + lane
            d = idxrow_v[pl.ds(cN + k * LANE, LANE)]
            hidx = ((c128 + (d >> 3)) << 4) + lane
            h = plsc.load_gather(head_v, [hidx])
            plsc.store_scatter(next_v, [cN + e16], h)
            plsc.store_scatter(head_v, [hidx], e16)
            return carry
        lax.fori_loop(0, N // LANE, body, 0, unroll=4)

    build(0)
    j_last = (s0 + nslab - 1) // TROWS

    @pl.when(j_last > j0)
    def _():
        build(1)

    def slab_jt(s):
        j = s // TROWS
        return j, s - j * TROWS

    def poke_slab(b, s, val):
        j, t = slab_jt(s)
        c = j - j0
        cN = c * N

        def chase(ev):
            for _ in range(2):
                live = ev >= 0
                evc = jnp.maximum(ev, 0)
                d = plsc.load_gather(idxrow_v, [cN + evc])
                plsc.store_scatter(bufs[b], [d & 7, evc], val, mask=live)
                nxt = plsc.load_gather(next_v, [cN + evc])
                ev = jnp.where(live, nxt, ev)
            return ev

        e0 = head_v[pl.ds((c * 128 + t) * 16, LANE)]
        lax.while_loop(lambda ev: jnp.max(ev) >= 0, chase, e0)

    def start_hbm(b, s):
        j, t = slab_jt(s)
        d0 = pl.multiple_of(t * 8, 8)
        pltpu.async_copy(bufs[b], out_hbm.at[j, pl.ds(d0, 8)], sems[b])

    def wait_hbm(b):
        pltpu.make_async_copy(bufs[b], out_hbm.at[0, pl.ds(0, 8)],
                              sems[b]).wait()

    for b in range(NBUF):
        poke_slab(b, s0 + b, on_vec)
        start_hbm(b, s0 + b)

    def step(i, c):
        for b in range(NBUF):
            g = 2 + i * NBUF + b

            @pl.when(g < nslab)
            def _():
                wait_hbm(b)
                poke_slab(b, s0 + g - NBUF, off_vec)
                poke_slab(b, s0 + g, on_vec)
                start_hbm(b, s0 + g)
        return c

    lax.fori_loop(0, NITER, step, 0)

    for b in range(NBUF):
        wait_hbm(b)


def kernel(indices, on_value, off_value):
    idx_t = indices.T.astype(jnp.int32)
    idx_t = jnp.pad(idx_t, ((0, 1), (0, 0))).reshape(-1)
    onoff = jnp.concatenate([
        jnp.full((LANE,), on_value, jnp.float32),
        jnp.full((LANE,), off_value, jnp.float32),
    ])
    mesh = plsc.VectorSubcoreMesh(
        core_axis_name="c", subcore_axis_name="s",
        num_cores=NUM_CORES, num_subcores=NUM_SUBCORES)
    out = pl.kernel(
        _onehot_sc_body,
        out_type=jax.ShapeDtypeStruct((J, DEPTH, N), jnp.float32),
        mesh=mesh,
        compiler_params=pltpu.CompilerParams(
            needs_layout_passes=False, use_tc_tiling_on_sc=True),
        scratch_types=(
            [pltpu.VMEM((2 * N,), jnp.int32),
             pltpu.VMEM((2 * LANE,), jnp.float32),
             pltpu.VMEM((2 * 128 * LANE,), jnp.int32),
             pltpu.VMEM((2 * N,), jnp.int32),
             pltpu.VMEM((8, N), jnp.float32),
             pltpu.VMEM((8, N), jnp.float32)]
            + [pltpu.SemaphoreType.DMA] * NBUF
        ),
    )(idx_t, onoff)
    return jnp.transpose(out, (2, 0, 1))
